# baseline (speedup 1.0000x reference)
import functools

import jax
import jax.numpy as jnp
from jax import lax
from jax.experimental import pallas as pl
from jax.experimental.pallas import tpu as pltpu
from jax.experimental.pallas import tpu_sc as plsc

_SQRT1_2 = 0.7071067811865476


def _gelu(x):
    return 0.5 * x * (1.0 + lax.erf(x * _SQRT1_2))


def _ln(x, g, b, eps=1e-5):
    mu = jnp.mean(x, axis=-1, keepdims=True)
    var = jnp.mean((x - mu) ** 2, axis=-1, keepdims=True)
    return (x - mu) / jnp.sqrt(var + eps) * g + b


def _enc_body(atom, ew, eb, g, bt, w1a, w1b, b1, xo, po, qo):
    h = _gelu(jnp.dot(atom[...], ew[...], preferred_element_type=jnp.float32)
              + eb[...])
    xn = _ln(h, g[...], bt[...])
    xo[...] = xn
    po[...] = jnp.dot(xn, w1a[...], preferred_element_type=jnp.float32).astype(jnp.bfloat16)
    qo[...] = (jnp.dot(xn, w1b[...], preferred_element_type=jnp.float32)
               + b1[...]).astype(jnp.bfloat16)


def _edge_body(e_total, be, gp, gq, attr, w1c, w2, mo):
    i = pl.program_id(0)
    gsum = (gp[...].astype(jnp.float32) + gq[...].astype(jnp.float32)
            + attr[...] * w1c[...])
    m = _gelu(gsum)
    m2 = jnp.dot(m, w2[...], preferred_element_type=jnp.float32)
    rowid = lax.broadcasted_iota(jnp.int32, m2.shape, 0) + i * be
    mo[...] = jnp.where(rowid < e_total, m2, 0.0)


def _final_body(x, agg, deg, b2, g, bt, w1a, w1b, b1, xo, po, qo):
    t = x[...] + jnp.concatenate([agg[0], agg[1]], axis=-1) + deg[...] * b2[...]
    xn = _ln(t, g[...], bt[...])
    xo[...] = xn
    po[...] = jnp.dot(xn, w1a[...], preferred_element_type=jnp.float32).astype(jnp.bfloat16)
    qo[...] = (jnp.dot(xn, w1b[...], preferred_element_type=jnp.float32)
               + b1[...]).astype(jnp.bfloat16)


def _final_last_body(x, agg, deg, b2, g, bt, xo):
    t = x[...] + jnp.concatenate([agg[0], agg[1]], axis=-1) + deg[...] * b2[...]
    xo[...] = _ln(t, g[...], bt[...])


def _deg_finish_body(degp, dego):
    dego[...] = degp[0, :, 0:1] + degp[1, :, 0:1]


def _readout_body(n_total, x, hw1, hb1, hw2, hb2, hw3, hb3, out, sm, mx):
    i = pl.program_id(0)
    last = pl.num_programs(0) - 1
    blk = x[...]
    bsum = jnp.sum(blk, axis=0, keepdims=True)
    bmax = jnp.max(blk, axis=0, keepdims=True)

    @pl.when(i == 0)
    def _():
        sm[...] = bsum
        mx[...] = bmax
        out[...] = jnp.zeros_like(out)

    @pl.when(i > 0)
    def _():
        sm[...] = sm[...] + bsum
        mx[...] = jnp.maximum(mx[...], bmax)

    @pl.when(i == last)
    def _():
        ge = jnp.concatenate([sm[...] * (1.0 / n_total), mx[...]], axis=1)
        h1 = _gelu(jnp.dot(ge, hw1[...], preferred_element_type=jnp.float32)
                   + hb1[...])
        h2 = _gelu(jnp.dot(h1, hw2[...], preferred_element_type=jnp.float32)
                   + hb2[...])
        out[...] = jnp.dot(h2, hw3[...], preferred_element_type=jnp.float32) + hb3[...]


def _full(shape):
    return pl.BlockSpec(shape, lambda i: (0,) * len(shape))


def _rows(bn, d):
    return pl.BlockSpec((bn, d), lambda i: (i, 0))


_CH = 128


def _sc_gather_fn(n_chunks, epw, nc, p_hbm, q_hbm, src_hbm, dst_hbm, gp_hbm,
                  gq_hbm, idx_s, idx_d, bufp, bufq, gsemp, gsemq, wsemp, wsemq):
    c = lax.axis_index("c")
    s = lax.axis_index("s")
    wid = s * nc + c
    cbase = wid * (epw // _CH)
    base = wid * epw
    pltpu.sync_copy(src_hbm.at[pl.ds(cbase, epw // _CH)], idx_s)
    pltpu.sync_copy(dst_hbm.at[pl.ds(cbase, epw // _CH)], idx_d)

    def g_issue(j, slot):
        pltpu.async_copy(p_hbm.at[idx_s.at[j]], bufp.at[slot], gsemp.at[slot])
        pltpu.async_copy(q_hbm.at[idx_d.at[j]], bufq.at[slot], gsemq.at[slot])

    def g_wait(j, slot):
        pltpu.make_async_copy(p_hbm.at[idx_s.at[j]], bufp.at[slot], gsemp.at[slot]).wait()
        pltpu.make_async_copy(q_hbm.at[idx_d.at[j]], bufq.at[slot], gsemq.at[slot]).wait()

    def w_issue(j, slot):
        pltpu.async_copy(bufp.at[slot], gp_hbm.at[pl.ds(base + j * _CH, _CH)], wsemp.at[slot])
        pltpu.async_copy(bufq.at[slot], gq_hbm.at[pl.ds(base + j * _CH, _CH)], wsemq.at[slot])

    def w_wait(j, slot):
        pltpu.make_async_copy(bufp.at[slot], gp_hbm.at[pl.ds(base + j * _CH, _CH)], wsemp.at[slot]).wait()
        pltpu.make_async_copy(bufq.at[slot], gq_hbm.at[pl.ds(base + j * _CH, _CH)], wsemq.at[slot]).wait()

    g_issue(0, 0)

    def body(i, _):
        slot = lax.rem(i, 2)
        other = 1 - slot

        @pl.when(i >= 1)
        def _():
            w_wait(i - 1, other)

        @pl.when(i + 1 < n_chunks)
        def _():
            g_issue(i + 1, other)

        g_wait(i, slot)
        w_issue(i, slot)
        return 0

    lax.fori_loop(0, n_chunks, body, 0)
    w_wait(n_chunks - 1, (n_chunks - 1) % 2)


def _sc_scatter_fn(n_chunks, epw, hh, rpt, m2_hbm, dst_hbm, zeros_hbm,
                   out_hbm, agg_sh, idx_d, buf, lsem, ssem):
    c = lax.axis_index("c")
    s = lax.axis_index("s")
    base = s * epw
    cbase = s * n_chunks

    pltpu.sync_copy(zeros_hbm.at[pl.ds(s * rpt, rpt)], agg_sh.at[pl.ds(s * rpt, rpt)])
    pltpu.sync_copy(dst_hbm.at[pl.ds(cbase, n_chunks)], idx_d)
    plsc.subcore_barrier()

    def l_issue(j, slot):
        pltpu.async_copy(m2_hbm.at[pl.ds(base + j * _CH, _CH), pl.ds(c * hh, hh)],
                         buf.at[slot], lsem.at[slot])

    def l_wait(j, slot):
        pltpu.make_async_copy(m2_hbm.at[pl.ds(base + j * _CH, _CH), pl.ds(c * hh, hh)],
                              buf.at[slot], lsem.at[slot]).wait()

    def s_issue(j, slot):
        pltpu.async_copy(buf.at[slot], agg_sh.at[idx_d.at[j]], ssem.at[slot], add=True)

    def s_wait(j, slot):
        pltpu.make_async_copy(buf.at[slot], agg_sh.at[idx_d.at[j]], ssem.at[slot]).wait()

    l_issue(0, 0)

    def body(i, _):
        slot = lax.rem(i, 2)
        other = 1 - slot

        @pl.when(i >= 1)
        def _():
            s_wait(i - 1, other)

        @pl.when(i + 1 < n_chunks)
        def _():
            l_issue(i + 1, other)

        l_wait(i, slot)
        s_issue(i, slot)
        return 0

    lax.fori_loop(0, n_chunks, body, 0)
    s_wait(n_chunks - 1, (n_chunks - 1) % 2)
    plsc.subcore_barrier()
    pltpu.sync_copy(agg_sh.at[pl.ds(s * rpt, rpt)], out_hbm.at[c, pl.ds(s * rpt, rpt)])


def _sc_deg_fn(n_chunks, epc, epw, rpt, dst_hbm, ones_hbm, zeros_hbm,
               out_hbm, deg_sh, idx_d, ones_v):
    c = lax.axis_index("c")
    s = lax.axis_index("s")
    base = c * epc + s * epw

    pltpu.sync_copy(zeros_hbm.at[pl.ds(s * rpt, rpt)], deg_sh.at[pl.ds(s * rpt, rpt)])
    pltpu.sync_copy(ones_hbm, ones_v)
    plsc.subcore_barrier()

    def body(i, _):
        off = base + i * _CH
        pltpu.sync_copy(dst_hbm.at[pl.ds(off, _CH)], idx_d)
        pltpu.sync_copy(ones_v, deg_sh.at[idx_d], add=True)
        return 0

    lax.fori_loop(0, n_chunks, body, 0)
    plsc.subcore_barrier()
    pltpu.sync_copy(deg_sh.at[pl.ds(s * rpt, rpt)], out_hbm.at[c, pl.ds(s * rpt, rpt)])


def kernel(atom_features, edge_index, edge_attr, params):
    n, adim = atom_features.shape
    e = edge_index.shape[1]
    h = params['enc']['w'].shape[1]
    h2 = 2 * h
    dh = params['head']['w2'].shape[1]
    dout = params['head']['w3'].shape[1]
    f32 = jnp.float32

    ncores, nsub = 2, 16
    nw = ncores * nsub
    epad = ((e + nw * _CH - 1) // (nw * _CH)) * (nw * _CH)
    epw = epad // nw
    epc = epad // ncores
    epw_sc = epc // nsub
    n_chunks = epw // _CH

    src = jnp.pad(edge_index[0], (0, epad - e)).reshape(epad // _CH, _CH)
    dst1 = jnp.pad(edge_index[1], (0, epad - e))
    dst = dst1.reshape(epad // _CH, _CH)
    attr_p = jnp.pad(edge_attr, ((0, epad - e), (0, 0)))

    npad = ((n + 8 * nsub - 1) // (8 * nsub)) * (8 * nsub)
    rpt = npad // nsub
    zeros_nh = jnp.zeros((npad, h // ncores), f32)
    zeros_n16 = jnp.zeros((npad, 16), f32)
    ones16 = jnp.zeros((_CH, 16), f32).at[:, 0].set(1.0)

    mesh = plsc.VectorSubcoreMesh(core_axis_name="c", subcore_axis_name="s")
    sc_params = pltpu.CompilerParams(use_tc_tiling_on_sc=False)

    bf16 = jnp.bfloat16
    sc_gather = pl.kernel(
        functools.partial(_sc_gather_fn, n_chunks, epw, ncores),
        out_type=(jax.ShapeDtypeStruct((epad, h2), bf16),
                  jax.ShapeDtypeStruct((epad, h2), bf16)),
        mesh=mesh,
        scratch_types=[
            pltpu.VMEM((n_chunks, _CH), jnp.int32),
            pltpu.VMEM((n_chunks, _CH), jnp.int32),
            pltpu.VMEM((2, _CH, h2), bf16),
            pltpu.VMEM((2, _CH, h2), bf16),
            pltpu.SemaphoreType.DMA((2,)),
            pltpu.SemaphoreType.DMA((2,)),
            pltpu.SemaphoreType.DMA((2,)),
            pltpu.SemaphoreType.DMA((2,)),
        ],
        compiler_params=sc_params,
    )

    hh = h // ncores
    nchunks_sc = epad // nsub // _CH
    sc_scatter = pl.kernel(
        functools.partial(_sc_scatter_fn, nchunks_sc, epad // nsub, hh, rpt),
        out_type=jax.ShapeDtypeStruct((ncores, npad, hh), f32),
        mesh=mesh,
        scratch_types=[
            pltpu.VMEM_SHARED((npad, hh), f32),
            pltpu.VMEM((nchunks_sc, _CH), jnp.int32),
            pltpu.VMEM((2, _CH, hh), f32),
            pltpu.SemaphoreType.DMA((2,)),
            pltpu.SemaphoreType.DMA((2,)),
        ],
        compiler_params=sc_params,
    )

    sc_deg = pl.kernel(
        functools.partial(_sc_deg_fn, epc // _CH // nsub, epc, epw_sc, rpt),
        out_type=jax.ShapeDtypeStruct((ncores, npad, 16), f32),
        mesh=mesh,
        scratch_types=[
            pltpu.VMEM_SHARED((npad, 16), f32),
            pltpu.VMEM((_CH,), jnp.int32),
            pltpu.VMEM((_CH, 16), f32),
        ],
        compiler_params=sc_params,
    )

    bn = 2000
    gn = n // bn
    be = 2048
    ge = epad // be

    enc = params['enc']
    lp0 = params['layers'][0]
    x, p, q = pl.pallas_call(
        _enc_body,
        grid=(gn,),
        in_specs=[_rows(bn, adim), _full((adim, h)), _full((1, h)),
                  _full((1, h)), _full((1, h)),
                  _full((h, h2)), _full((h, h2)), _full((1, h2))],
        out_specs=[_rows(bn, h), _rows(bn, h2), _rows(bn, h2)],
        out_shape=[jax.ShapeDtypeStruct((n, h), f32),
                   jax.ShapeDtypeStruct((n, h2), bf16),
                   jax.ShapeDtypeStruct((n, h2), bf16)],
    )(atom_features, enc['w'], enc['b'].reshape(1, h),
      enc['g'].reshape(1, h), enc['bt'].reshape(1, h),
      lp0['w1'][:h], lp0['w1'][h:h2], lp0['b1'].reshape(1, h2))

    degp = sc_deg(dst1, ones16, zeros_n16)
    deg = pl.pallas_call(
        _deg_finish_body,
        grid=(gn,),
        in_specs=[pl.BlockSpec((ncores, bn, 16), lambda i: (0, i, 0))],
        out_specs=_rows(bn, 1),
        out_shape=jax.ShapeDtypeStruct((n, 1), f32),
    )(degp)

    n_layers = len(params['layers'])
    for li, lp in enumerate(params['layers']):
        gp, gq = sc_gather(p, q, src, dst)
        m2 = pl.pallas_call(
            functools.partial(_edge_body, e, be),
            grid=(ge,),
            in_specs=[_rows(be, h2), _rows(be, h2), _rows(be, 1),
                      _full((1, h2)), _full((h2, h))],
            out_specs=_rows(be, h),
            out_shape=jax.ShapeDtypeStruct((epad, h), f32),
        )(gp, gq, attr_p, lp['w1'][h2:h2 + 1], lp['w2'])
        aggp = sc_scatter(m2, dst, zeros_nh)

        common = (x, aggp, deg, lp['b2'].reshape(1, h),
                  lp['g'].reshape(1, h), lp['bt'].reshape(1, h))
        common_specs = [_rows(bn, h), pl.BlockSpec((ncores, bn, h // ncores), lambda i: (0, i, 0)),
                        _rows(bn, 1), _full((1, h)), _full((1, h)), _full((1, h))]
        if li + 1 < n_layers:
            nxt = params['layers'][li + 1]
            x, p, q = pl.pallas_call(
                _final_body,
                grid=(gn,),
                in_specs=common_specs + [_full((h, h2)), _full((h, h2)), _full((1, h2))],
                out_specs=[_rows(bn, h), _rows(bn, h2), _rows(bn, h2)],
                out_shape=[jax.ShapeDtypeStruct((n, h), f32),
                           jax.ShapeDtypeStruct((n, h2), bf16),
                           jax.ShapeDtypeStruct((n, h2), bf16)],
            )(*common, nxt['w1'][:h], nxt['w1'][h:h2], nxt['b1'].reshape(1, h2))
        else:
            x = pl.pallas_call(
                _final_last_body,
                grid=(gn,),
                in_specs=common_specs,
                out_specs=_rows(bn, h),
                out_shape=jax.ShapeDtypeStruct((n, h), f32),
            )(*common)

    hp = params['head']
    out = pl.pallas_call(
        functools.partial(_readout_body, n),
        grid=(gn,),
        in_specs=[_rows(bn, h), _full((h2, h)), _full((1, h)),
                  _full((h, dh)), _full((1, dh)),
                  _full((dh, dout)), _full((1, dout))],
        out_specs=_full((1, dout)),
        out_shape=jax.ShapeDtypeStruct((1, dout), f32),
        scratch_shapes=[pltpu.VMEM((1, h), f32), pltpu.VMEM((1, h), f32)],
    )(x, hp['w1'], hp['b1'].reshape(1, h), hp['w2'], hp['b2'].reshape(1, dh),
      hp['w3'], hp['b3'].reshape(1, dout))
    return out


# speedup vs baseline: 1.1800x; 1.1800x over previous
import functools

import jax
import jax.numpy as jnp
from jax import lax
from jax.experimental import pallas as pl
from jax.experimental.pallas import tpu as pltpu
from jax.experimental.pallas import tpu_sc as plsc

_SQRT1_2 = 0.7071067811865476


def _gelu(x):
    return 0.5 * x * (1.0 + lax.erf(x * _SQRT1_2))


def _ln(x, g, b, eps=1e-5):
    mu = jnp.mean(x, axis=-1, keepdims=True)
    var = jnp.mean((x - mu) ** 2, axis=-1, keepdims=True)
    return (x - mu) / jnp.sqrt(var + eps) * g + b


def _enc_body(atom, ew, eb, g, bt, w1a, w1b, b1, xo, po, qo):
    h = _gelu(jnp.dot(atom[...], ew[...], preferred_element_type=jnp.float32)
              + eb[...])
    xn = _ln(h, g[...], bt[...])
    xo[...] = xn
    po[...] = jnp.dot(xn, w1a[...], preferred_element_type=jnp.float32)
    qo[...] = jnp.dot(xn, w1b[...], preferred_element_type=jnp.float32) + b1[...]


def _edge_body(e_total, be, gp, gq, attr, w1c, w2, mo):
    i = pl.program_id(0)
    gsum = gp[...] + gq[...] + attr[...] * w1c[...]
    m = _gelu(gsum)
    m2 = jnp.dot(m, w2[...], preferred_element_type=jnp.float32)
    rowid = lax.broadcasted_iota(jnp.int32, m2.shape, 0) + i * be
    mo[...] = jnp.where(rowid < e_total, m2, 0.0)


def _final_body(x, agg, deg, b2, g, bt, w1a, w1b, b1, xo, po, qo):
    t = x[...] + jnp.concatenate([agg[0], agg[1]], axis=-1) + deg[...] * b2[...]
    xn = _ln(t, g[...], bt[...])
    xo[...] = xn
    po[...] = jnp.dot(xn, w1a[...], preferred_element_type=jnp.float32)
    qo[...] = jnp.dot(xn, w1b[...], preferred_element_type=jnp.float32) + b1[...]


def _final_last_body(x, agg, deg, b2, g, bt, xo):
    t = x[...] + jnp.concatenate([agg[0], agg[1]], axis=-1) + deg[...] * b2[...]
    xo[...] = _ln(t, g[...], bt[...])


def _deg_finish_body(degp, dego):
    dego[...] = degp[0, :, 0:1] + degp[1, :, 0:1]


def _readout_body(n_total, x, hw1, hb1, hw2, hb2, hw3, hb3, out, sm, mx):
    i = pl.program_id(0)
    last = pl.num_programs(0) - 1
    blk = x[...]
    bsum = jnp.sum(blk, axis=0, keepdims=True)
    bmax = jnp.max(blk, axis=0, keepdims=True)

    @pl.when(i == 0)
    def _():
        sm[...] = bsum
        mx[...] = bmax
        out[...] = jnp.zeros_like(out)

    @pl.when(i > 0)
    def _():
        sm[...] = sm[...] + bsum
        mx[...] = jnp.maximum(mx[...], bmax)

    @pl.when(i == last)
    def _():
        ge = jnp.concatenate([sm[...] * (1.0 / n_total), mx[...]], axis=1)
        h1 = _gelu(jnp.dot(ge, hw1[...], preferred_element_type=jnp.float32)
                   + hb1[...])
        h2 = _gelu(jnp.dot(h1, hw2[...], preferred_element_type=jnp.float32)
                   + hb2[...])
        out[...] = jnp.dot(h2, hw3[...], preferred_element_type=jnp.float32) + hb3[...]


def _full(shape):
    return pl.BlockSpec(shape, lambda i: (0,) * len(shape))


def _rows(bn, d):
    return pl.BlockSpec((bn, d), lambda i: (i, 0))


_CH = 128


def _sc_gather_fn(n_chunks, epw, gch, nc, p_hbm, q_hbm, src_hbm, dst_hbm,
                  gp_hbm, gq_hbm, idx_s, idx_d, bufp, bufq, gsemp, gsemq,
                  wsemp, wsemq):
    c = lax.axis_index("c")
    s = lax.axis_index("s")
    wid = s * nc + c
    cbase = wid * n_chunks
    base = wid * epw
    pltpu.sync_copy(src_hbm.at[pl.ds(cbase, n_chunks)], idx_s)
    pltpu.sync_copy(dst_hbm.at[pl.ds(cbase, n_chunks)], idx_d)

    def g_issue(j, slot):
        pltpu.async_copy(p_hbm.at[idx_s.at[j]], bufp.at[slot], gsemp.at[slot])
        pltpu.async_copy(q_hbm.at[idx_d.at[j]], bufq.at[slot], gsemq.at[slot])

    def g_wait(j, slot):
        pltpu.make_async_copy(p_hbm.at[idx_s.at[j]], bufp.at[slot], gsemp.at[slot]).wait()
        pltpu.make_async_copy(q_hbm.at[idx_d.at[j]], bufq.at[slot], gsemq.at[slot]).wait()

    def w_issue(j, slot):
        pltpu.async_copy(bufp.at[slot], gp_hbm.at[pl.ds(base + j * gch, gch)], wsemp.at[slot])
        pltpu.async_copy(bufq.at[slot], gq_hbm.at[pl.ds(base + j * gch, gch)], wsemq.at[slot])

    def w_wait(j, slot):
        pltpu.make_async_copy(bufp.at[slot], gp_hbm.at[pl.ds(base + j * gch, gch)], wsemp.at[slot]).wait()
        pltpu.make_async_copy(bufq.at[slot], gq_hbm.at[pl.ds(base + j * gch, gch)], wsemq.at[slot]).wait()

    g_issue(0, 0)

    def body(i, _):
        slot = lax.rem(i, 2)
        other = 1 - slot

        @pl.when(i >= 1)
        def _():
            w_wait(i - 1, other)

        @pl.when(i + 1 < n_chunks)
        def _():
            g_issue(i + 1, other)

        g_wait(i, slot)
        w_issue(i, slot)
        return 0

    lax.fori_loop(0, n_chunks, body, 0)
    w_wait(n_chunks - 1, (n_chunks - 1) % 2)


def _sc_scatter_fn(n_chunks, epw, hh, rpt, m2_hbm, dst_hbm, zeros_hbm,
                   out_hbm, agg_sh, idx_d, buf, lsem, ssem):
    c = lax.axis_index("c")
    s = lax.axis_index("s")
    base = s * epw
    cbase = s * n_chunks

    pltpu.sync_copy(zeros_hbm.at[pl.ds(s * rpt, rpt)], agg_sh.at[pl.ds(s * rpt, rpt)])
    pltpu.sync_copy(dst_hbm.at[pl.ds(cbase, n_chunks)], idx_d)
    plsc.subcore_barrier()

    def l_issue(j, slot):
        pltpu.async_copy(m2_hbm.at[pl.ds(base + j * _CH, _CH), pl.ds(c * hh, hh)],
                         buf.at[slot], lsem.at[slot])

    def l_wait(j, slot):
        pltpu.make_async_copy(m2_hbm.at[pl.ds(base + j * _CH, _CH), pl.ds(c * hh, hh)],
                              buf.at[slot], lsem.at[slot]).wait()

    def s_issue(j, slot):
        pltpu.async_copy(buf.at[slot], agg_sh.at[idx_d.at[j]], ssem.at[slot], add=True)

    def s_wait(j, slot):
        pltpu.make_async_copy(buf.at[slot], agg_sh.at[idx_d.at[j]], ssem.at[slot]).wait()

    l_issue(0, 0)

    def body(i, _):
        slot = lax.rem(i, 2)
        other = 1 - slot

        @pl.when(i >= 1)
        def _():
            s_wait(i - 1, other)

        @pl.when(i + 1 < n_chunks)
        def _():
            l_issue(i + 1, other)

        l_wait(i, slot)
        s_issue(i, slot)
        return 0

    lax.fori_loop(0, n_chunks, body, 0)
    s_wait(n_chunks - 1, (n_chunks - 1) % 2)
    plsc.subcore_barrier()
    pltpu.sync_copy(agg_sh.at[pl.ds(s * rpt, rpt)], out_hbm.at[c, pl.ds(s * rpt, rpt)])


def _sc_deg_fn(n_chunks, epc, epw, rpt, dst_hbm, ones_hbm, zeros_hbm,
               out_hbm, deg_sh, idx_d, ones_v):
    c = lax.axis_index("c")
    s = lax.axis_index("s")
    base = c * epc + s * epw

    pltpu.sync_copy(zeros_hbm.at[pl.ds(s * rpt, rpt)], deg_sh.at[pl.ds(s * rpt, rpt)])
    pltpu.sync_copy(ones_hbm, ones_v)
    plsc.subcore_barrier()

    def body(i, _):
        off = base + i * _CH
        pltpu.sync_copy(dst_hbm.at[pl.ds(off, _CH)], idx_d)
        pltpu.sync_copy(ones_v, deg_sh.at[idx_d], add=True)
        return 0

    lax.fori_loop(0, n_chunks, body, 0)
    plsc.subcore_barrier()
    pltpu.sync_copy(deg_sh.at[pl.ds(s * rpt, rpt)], out_hbm.at[c, pl.ds(s * rpt, rpt)])


def kernel(atom_features, edge_index, edge_attr, params):
    n, adim = atom_features.shape
    e = edge_index.shape[1]
    h = params['enc']['w'].shape[1]
    h2 = 2 * h
    dh = params['head']['w2'].shape[1]
    dout = params['head']['w3'].shape[1]
    f32 = jnp.float32

    ncores, nsub = 2, 16
    nw = ncores * nsub
    epad = ((e + nw * _CH - 1) // (nw * _CH)) * (nw * _CH)
    epw = epad // nw
    epc = epad // ncores
    epw_sc = epc // nsub
    n_chunks = epw // _CH

    gch = 64
    src = jnp.pad(edge_index[0], (0, epad - e)).reshape(epad // gch, gch)
    dst1 = jnp.pad(edge_index[1], (0, epad - e))
    dstg = dst1.reshape(epad // gch, gch)
    dst = dst1.reshape(epad // _CH, _CH)
    attr_p = jnp.pad(edge_attr, ((0, epad - e), (0, 0)))

    npad = ((n + 8 * nsub - 1) // (8 * nsub)) * (8 * nsub)
    rpt = npad // nsub
    zeros_nh = jnp.zeros((npad, h // ncores), f32)
    zeros_n16 = jnp.zeros((npad, 16), f32)
    ones16 = jnp.zeros((_CH, 16), f32).at[:, 0].set(1.0)

    mesh = plsc.VectorSubcoreMesh(core_axis_name="c", subcore_axis_name="s")
    sc_params = pltpu.CompilerParams(use_tc_tiling_on_sc=False)

    ng = epw // gch
    sc_gather = pl.kernel(
        functools.partial(_sc_gather_fn, ng, epw, gch, ncores),
        out_type=(jax.ShapeDtypeStruct((epad, h2), f32),
                  jax.ShapeDtypeStruct((epad, h2), f32)),
        mesh=mesh,
        scratch_types=[
            pltpu.VMEM((ng, gch), jnp.int32),
            pltpu.VMEM((ng, gch), jnp.int32),
            pltpu.VMEM((2, gch, h2), f32),
            pltpu.VMEM((2, gch, h2), f32),
            pltpu.SemaphoreType.DMA((2,)),
            pltpu.SemaphoreType.DMA((2,)),
            pltpu.SemaphoreType.DMA((2,)),
            pltpu.SemaphoreType.DMA((2,)),
        ],
        compiler_params=sc_params,
    )

    hh = h // ncores
    nchunks_sc = epad // nsub // _CH
    sc_scatter = pl.kernel(
        functools.partial(_sc_scatter_fn, nchunks_sc, epad // nsub, hh, rpt),
        out_type=jax.ShapeDtypeStruct((ncores, npad, hh), f32),
        mesh=mesh,
        scratch_types=[
            pltpu.VMEM_SHARED((npad, hh), f32),
            pltpu.VMEM((nchunks_sc, _CH), jnp.int32),
            pltpu.VMEM((2, _CH, hh), f32),
            pltpu.SemaphoreType.DMA((2,)),
            pltpu.SemaphoreType.DMA((2,)),
        ],
        compiler_params=sc_params,
    )

    sc_deg = pl.kernel(
        functools.partial(_sc_deg_fn, epc // _CH // nsub, epc, epw_sc, rpt),
        out_type=jax.ShapeDtypeStruct((ncores, npad, 16), f32),
        mesh=mesh,
        scratch_types=[
            pltpu.VMEM_SHARED((npad, 16), f32),
            pltpu.VMEM((_CH,), jnp.int32),
            pltpu.VMEM((_CH, 16), f32),
        ],
        compiler_params=sc_params,
    )

    bn = 2000
    gn = n // bn
    be = 2048
    ge = epad // be

    enc = params['enc']
    lp0 = params['layers'][0]
    x, p, q = pl.pallas_call(
        _enc_body,
        grid=(gn,),
        in_specs=[_rows(bn, adim), _full((adim, h)), _full((1, h)),
                  _full((1, h)), _full((1, h)),
                  _full((h, h2)), _full((h, h2)), _full((1, h2))],
        out_specs=[_rows(bn, h), _rows(bn, h2), _rows(bn, h2)],
        out_shape=[jax.ShapeDtypeStruct((n, h), f32),
                   jax.ShapeDtypeStruct((n, h2), f32),
                   jax.ShapeDtypeStruct((n, h2), f32)],
    )(atom_features, enc['w'], enc['b'].reshape(1, h),
      enc['g'].reshape(1, h), enc['bt'].reshape(1, h),
      lp0['w1'][:h], lp0['w1'][h:h2], lp0['b1'].reshape(1, h2))

    degp = sc_deg(dst1, ones16, zeros_n16)
    deg = pl.pallas_call(
        _deg_finish_body,
        grid=(gn,),
        in_specs=[pl.BlockSpec((ncores, bn, 16), lambda i: (0, i, 0))],
        out_specs=_rows(bn, 1),
        out_shape=jax.ShapeDtypeStruct((n, 1), f32),
    )(degp)

    n_layers = len(params['layers'])
    for li, lp in enumerate(params['layers']):
        gp, gq = sc_gather(p, q, src, dstg)
        m2 = pl.pallas_call(
            functools.partial(_edge_body, e, be),
            grid=(ge,),
            in_specs=[_rows(be, h2), _rows(be, h2), _rows(be, 1),
                      _full((1, h2)), _full((h2, h))],
            out_specs=_rows(be, h),
            out_shape=jax.ShapeDtypeStruct((epad, h), f32),
        )(gp, gq, attr_p, lp['w1'][h2:h2 + 1], lp['w2'])
        aggp = sc_scatter(m2, dst, zeros_nh)

        common = (x, aggp, deg, lp['b2'].reshape(1, h),
                  lp['g'].reshape(1, h), lp['bt'].reshape(1, h))
        common_specs = [_rows(bn, h), pl.BlockSpec((ncores, bn, h // ncores), lambda i: (0, i, 0)),
                        _rows(bn, 1), _full((1, h)), _full((1, h)), _full((1, h))]
        if li + 1 < n_layers:
            nxt = params['layers'][li + 1]
            x, p, q = pl.pallas_call(
                _final_body,
                grid=(gn,),
                in_specs=common_specs + [_full((h, h2)), _full((h, h2)), _full((1, h2))],
                out_specs=[_rows(bn, h), _rows(bn, h2), _rows(bn, h2)],
                out_shape=[jax.ShapeDtypeStruct((n, h), f32),
                           jax.ShapeDtypeStruct((n, h2), f32),
                           jax.ShapeDtypeStruct((n, h2), f32)],
            )(*common, nxt['w1'][:h], nxt['w1'][h:h2], nxt['b1'].reshape(1, h2))
        else:
            x = pl.pallas_call(
                _final_last_body,
                grid=(gn,),
                in_specs=common_specs,
                out_specs=_rows(bn, h),
                out_shape=jax.ShapeDtypeStruct((n, h), f32),
            )(*common)

    hp = params['head']
    out = pl.pallas_call(
        functools.partial(_readout_body, n),
        grid=(gn,),
        in_specs=[_rows(bn, h), _full((h2, h)), _full((1, h)),
                  _full((h, dh)), _full((1, dh)),
                  _full((dh, dout)), _full((1, dout))],
        out_specs=_full((1, dout)),
        out_shape=jax.ShapeDtypeStruct((1, dout), f32),
        scratch_shapes=[pltpu.VMEM((1, h), f32), pltpu.VMEM((1, h), f32)],
    )(x, hp['w1'], hp['b1'].reshape(1, h), hp['w2'], hp['b2'].reshape(1, dh),
      hp['w3'], hp['b3'].reshape(1, dout))
    return out


# speedup vs baseline: 1.7620x; 1.4933x over previous
import functools

import jax
import jax.numpy as jnp
from jax import lax
from jax.experimental import pallas as pl
from jax.experimental.pallas import tpu as pltpu
from jax.experimental.pallas import tpu_sc as plsc

_SQRT1_2 = 0.7071067811865476


def _gelu(x):
    return 0.5 * x * (1.0 + lax.erf(x * _SQRT1_2))


def _ln(x, g, b, eps=1e-5):
    mu = jnp.mean(x, axis=-1, keepdims=True)
    var = jnp.mean((x - mu) ** 2, axis=-1, keepdims=True)
    return (x - mu) / jnp.sqrt(var + eps) * g + b


def _enc_body(atom, ew, eb, g, bt, w1a, w1b, b1, xo, po, qo):
    h = _gelu(jnp.dot(atom[...], ew[...], preferred_element_type=jnp.float32)
              + eb[...])
    xn = _ln(h, g[...], bt[...])
    xo[...] = xn
    po[...] = jnp.dot(xn, w1a[...], preferred_element_type=jnp.float32)
    qo[...] = jnp.dot(xn, w1b[...], preferred_element_type=jnp.float32) + b1[...]


def _edge_body(e_total, be, gp, gq, attr, w1c, w2, mo):
    i = pl.program_id(0)
    gsum = gp[...] + gq[...] + attr[...] * w1c[...]
    m = _gelu(gsum)
    m2 = jnp.dot(m, w2[...], preferred_element_type=jnp.float32)
    rowid = lax.broadcasted_iota(jnp.int32, m2.shape, 0) + i * be
    mo[...] = jnp.where(rowid < e_total, m2, 0.0)


def _final_body(x, agg, deg, b2, g, bt, w1a, w1b, b1, xo, po, qo):
    t = x[...] + jnp.concatenate([agg[0], agg[1]], axis=-1) + deg[...] * b2[...]
    xn = _ln(t, g[...], bt[...])
    xo[...] = xn
    po[...] = jnp.dot(xn, w1a[...], preferred_element_type=jnp.float32)
    qo[...] = jnp.dot(xn, w1b[...], preferred_element_type=jnp.float32) + b1[...]


def _final_last_body(x, agg, deg, b2, g, bt, xo):
    t = x[...] + jnp.concatenate([agg[0], agg[1]], axis=-1) + deg[...] * b2[...]
    xo[...] = _ln(t, g[...], bt[...])


def _deg_finish_body(degp, dego):
    dego[...] = degp[0, :, 0:1] + degp[1, :, 0:1]


def _readout_body(n_total, x, hw1, hb1, hw2, hb2, hw3, hb3, out, sm, mx):
    i = pl.program_id(0)
    last = pl.num_programs(0) - 1
    blk = x[...]
    bsum = jnp.sum(blk, axis=0, keepdims=True)
    bmax = jnp.max(blk, axis=0, keepdims=True)

    @pl.when(i == 0)
    def _():
        sm[...] = bsum
        mx[...] = bmax
        out[...] = jnp.zeros_like(out)

    @pl.when(i > 0)
    def _():
        sm[...] = sm[...] + bsum
        mx[...] = jnp.maximum(mx[...], bmax)

    @pl.when(i == last)
    def _():
        ge = jnp.concatenate([sm[...] * (1.0 / n_total), mx[...]], axis=1)
        h1 = _gelu(jnp.dot(ge, hw1[...], preferred_element_type=jnp.float32)
                   + hb1[...])
        h2 = _gelu(jnp.dot(h1, hw2[...], preferred_element_type=jnp.float32)
                   + hb2[...])
        out[...] = jnp.dot(h2, hw3[...], preferred_element_type=jnp.float32) + hb3[...]


def _full(shape):
    return pl.BlockSpec(shape, lambda i: (0,) * len(shape))


def _rows(bn, d):
    return pl.BlockSpec((bn, d), lambda i: (i, 0))


_CH = 128


def _sc_gather_fn(n_chunks, epw, gch, nc, p_hbm, q_hbm, src_hbm, dst_hbm,
                  gp_hbm, gq_hbm, idx_s, idx_d, bufp, bufq, gsemp, gsemq,
                  wsemp, wsemq):
    c = lax.axis_index("c")
    s = lax.axis_index("s")
    wid = s * nc + c
    cbase = wid * n_chunks
    base = wid * epw
    pltpu.sync_copy(src_hbm.at[pl.ds(cbase, n_chunks)], idx_s)
    pltpu.sync_copy(dst_hbm.at[pl.ds(cbase, n_chunks)], idx_d)

    def g_issue(j, slot):
        pltpu.async_copy(p_hbm.at[idx_s.at[j]], bufp.at[slot], gsemp.at[slot])
        pltpu.async_copy(q_hbm.at[idx_d.at[j]], bufq.at[slot], gsemq.at[slot])

    def g_wait(j, slot):
        pltpu.make_async_copy(p_hbm.at[idx_s.at[j]], bufp.at[slot], gsemp.at[slot]).wait()
        pltpu.make_async_copy(q_hbm.at[idx_d.at[j]], bufq.at[slot], gsemq.at[slot]).wait()

    def w_issue(j, slot):
        pltpu.async_copy(bufp.at[slot], gp_hbm.at[pl.ds(base + j * gch, gch)], wsemp.at[slot])
        pltpu.async_copy(bufq.at[slot], gq_hbm.at[pl.ds(base + j * gch, gch)], wsemq.at[slot])

    def w_wait(j, slot):
        pltpu.make_async_copy(bufp.at[slot], gp_hbm.at[pl.ds(base + j * gch, gch)], wsemp.at[slot]).wait()
        pltpu.make_async_copy(bufq.at[slot], gq_hbm.at[pl.ds(base + j * gch, gch)], wsemq.at[slot]).wait()

    g_issue(0, 0)

    def body(i, _):
        slot = lax.rem(i, 2)
        other = 1 - slot

        @pl.when(i >= 1)
        def _():
            w_wait(i - 1, other)

        @pl.when(i + 1 < n_chunks)
        def _():
            g_issue(i + 1, other)

        g_wait(i, slot)
        w_issue(i, slot)
        return 0

    lax.fori_loop(0, n_chunks, body, 0)
    w_wait(n_chunks - 1, (n_chunks - 1) % 2)


def _sc_scatter_fn(n_chunks, epw, hh, rpt, m2_hbm, dst_hbm, zeros_hbm,
                   out_hbm, agg_sh, idx_d, buf, lsem, ssem):
    c = lax.axis_index("c")
    s = lax.axis_index("s")
    base = s * epw
    cbase = s * n_chunks

    pltpu.sync_copy(zeros_hbm.at[pl.ds(s * rpt, rpt)], agg_sh.at[pl.ds(s * rpt, rpt)])
    pltpu.sync_copy(dst_hbm.at[pl.ds(cbase, n_chunks)], idx_d)
    plsc.subcore_barrier()

    def l_issue(j, slot):
        pltpu.async_copy(m2_hbm.at[pl.ds(base + j * _CH, _CH), pl.ds(c * hh, hh)],
                         buf.at[slot], lsem.at[slot])

    def l_wait(j, slot):
        pltpu.make_async_copy(m2_hbm.at[pl.ds(base + j * _CH, _CH), pl.ds(c * hh, hh)],
                              buf.at[slot], lsem.at[slot]).wait()

    def s_issue(j, slot):
        pltpu.async_copy(buf.at[slot], agg_sh.at[idx_d.at[j]], ssem.at[slot], add=True)

    def s_wait(j, slot):
        pltpu.make_async_copy(buf.at[slot], agg_sh.at[idx_d.at[j]], ssem.at[slot]).wait()

    l_issue(0, 0)

    def body(i, _):
        slot = lax.rem(i, 2)
        other = 1 - slot

        @pl.when(i >= 1)
        def _():
            s_wait(i - 1, other)

        @pl.when(i + 1 < n_chunks)
        def _():
            l_issue(i + 1, other)

        l_wait(i, slot)
        s_issue(i, slot)
        return 0

    lax.fori_loop(0, n_chunks, body, 0)
    s_wait(n_chunks - 1, (n_chunks - 1) % 2)
    plsc.subcore_barrier()
    pltpu.sync_copy(agg_sh.at[pl.ds(s * rpt, rpt)], out_hbm.at[c, pl.ds(s * rpt, rpt)])


def _sc_deg_fn(n_chunks, epc, epw, rpt, dst_hbm, ones_hbm, zeros_hbm,
               out_hbm, deg_sh, idx_d, ones_v):
    c = lax.axis_index("c")
    s = lax.axis_index("s")
    base = c * epc + s * epw

    pltpu.sync_copy(zeros_hbm.at[pl.ds(s * rpt, rpt)], deg_sh.at[pl.ds(s * rpt, rpt)])
    pltpu.sync_copy(ones_hbm, ones_v)
    plsc.subcore_barrier()

    def body(i, _):
        off = base + i * _CH
        pltpu.sync_copy(dst_hbm.at[pl.ds(off, _CH)], idx_d)
        pltpu.sync_copy(ones_v, deg_sh.at[idx_d], add=True)
        return 0

    lax.fori_loop(0, n_chunks, body, 0)
    plsc.subcore_barrier()
    pltpu.sync_copy(deg_sh.at[pl.ds(s * rpt, rpt)], out_hbm.at[c, pl.ds(s * rpt, rpt)])


def kernel(atom_features, edge_index, edge_attr, params):
    n, adim = atom_features.shape
    e = edge_index.shape[1]
    h = params['enc']['w'].shape[1]
    h2 = 2 * h
    dh = params['head']['w2'].shape[1]
    dout = params['head']['w3'].shape[1]
    f32 = jnp.float32

    ncores, nsub = 2, 16
    nw = ncores * nsub
    epad = ((e + nw * _CH - 1) // (nw * _CH)) * (nw * _CH)
    epw = epad // nw
    epc = epad // ncores
    epw_sc = epc // nsub
    n_chunks = epw // _CH

    gch = 64
    src = jnp.pad(edge_index[0], (0, epad - e)).reshape(epad // gch, gch)
    dst1 = jnp.pad(edge_index[1], (0, epad - e))
    dstg = dst1.reshape(epad // gch, gch)
    dst = dst1.reshape(epad // _CH, _CH)
    attr_p = jnp.pad(edge_attr, ((0, epad - e), (0, 0)))

    npad = ((n + 8 * nsub - 1) // (8 * nsub)) * (8 * nsub)
    rpt = npad // nsub
    zeros_nh = jnp.zeros((npad, h // ncores), f32)
    zeros_n16 = jnp.zeros((npad, 16), f32)
    ones16 = jnp.zeros((_CH, 16), f32).at[:, 0].set(1.0)

    mesh = plsc.VectorSubcoreMesh(core_axis_name="c", subcore_axis_name="s")
    sc_params = pltpu.CompilerParams(use_tc_tiling_on_sc=False)

    ng = epw // gch
    sc_gather = pl.kernel(
        functools.partial(_sc_gather_fn, ng, epw, gch, ncores),
        out_type=(jax.ShapeDtypeStruct((epad, h2), f32),
                  jax.ShapeDtypeStruct((epad, h2), f32)),
        mesh=mesh,
        scratch_types=[
            pltpu.VMEM((ng, gch), jnp.int32),
            pltpu.VMEM((ng, gch), jnp.int32),
            pltpu.VMEM((2, gch, h2), f32),
            pltpu.VMEM((2, gch, h2), f32),
            pltpu.SemaphoreType.DMA((2,)),
            pltpu.SemaphoreType.DMA((2,)),
            pltpu.SemaphoreType.DMA((2,)),
            pltpu.SemaphoreType.DMA((2,)),
        ],
        compiler_params=pltpu.CompilerParams(use_tc_tiling_on_sc=True),
    )

    hh = h // ncores
    nchunks_sc = epad // nsub // _CH
    sc_scatter = pl.kernel(
        functools.partial(_sc_scatter_fn, nchunks_sc, epad // nsub, hh, rpt),
        out_type=jax.ShapeDtypeStruct((ncores, npad, hh), f32),
        mesh=mesh,
        scratch_types=[
            pltpu.VMEM_SHARED((npad, hh), f32),
            pltpu.VMEM((nchunks_sc, _CH), jnp.int32),
            pltpu.VMEM((2, _CH, hh), f32),
            pltpu.SemaphoreType.DMA((2,)),
            pltpu.SemaphoreType.DMA((2,)),
        ],
        compiler_params=sc_params,
    )

    sc_deg = pl.kernel(
        functools.partial(_sc_deg_fn, epc // _CH // nsub, epc, epw_sc, rpt),
        out_type=jax.ShapeDtypeStruct((ncores, npad, 16), f32),
        mesh=mesh,
        scratch_types=[
            pltpu.VMEM_SHARED((npad, 16), f32),
            pltpu.VMEM((_CH,), jnp.int32),
            pltpu.VMEM((_CH, 16), f32),
        ],
        compiler_params=sc_params,
    )

    bn = 2000
    gn = n // bn
    be = 2048
    ge = epad // be

    enc = params['enc']
    lp0 = params['layers'][0]
    x, p, q = pl.pallas_call(
        _enc_body,
        grid=(gn,),
        in_specs=[_rows(bn, adim), _full((adim, h)), _full((1, h)),
                  _full((1, h)), _full((1, h)),
                  _full((h, h2)), _full((h, h2)), _full((1, h2))],
        out_specs=[_rows(bn, h), _rows(bn, h2), _rows(bn, h2)],
        out_shape=[jax.ShapeDtypeStruct((n, h), f32),
                   jax.ShapeDtypeStruct((n, h2), f32),
                   jax.ShapeDtypeStruct((n, h2), f32)],
    )(atom_features, enc['w'], enc['b'].reshape(1, h),
      enc['g'].reshape(1, h), enc['bt'].reshape(1, h),
      lp0['w1'][:h], lp0['w1'][h:h2], lp0['b1'].reshape(1, h2))

    degp = sc_deg(dst1, ones16, zeros_n16)
    deg = pl.pallas_call(
        _deg_finish_body,
        grid=(gn,),
        in_specs=[pl.BlockSpec((ncores, bn, 16), lambda i: (0, i, 0))],
        out_specs=_rows(bn, 1),
        out_shape=jax.ShapeDtypeStruct((n, 1), f32),
    )(degp)

    n_layers = len(params['layers'])
    for li, lp in enumerate(params['layers']):
        gp, gq = sc_gather(p, q, src, dstg)
        m2 = pl.pallas_call(
            functools.partial(_edge_body, e, be),
            grid=(ge,),
            in_specs=[_rows(be, h2), _rows(be, h2), _rows(be, 1),
                      _full((1, h2)), _full((h2, h))],
            out_specs=_rows(be, h),
            out_shape=jax.ShapeDtypeStruct((epad, h), f32),
        )(gp, gq, attr_p, lp['w1'][h2:h2 + 1], lp['w2'])
        aggp = sc_scatter(m2, dst, zeros_nh)

        common = (x, aggp, deg, lp['b2'].reshape(1, h),
                  lp['g'].reshape(1, h), lp['bt'].reshape(1, h))
        common_specs = [_rows(bn, h), pl.BlockSpec((ncores, bn, h // ncores), lambda i: (0, i, 0)),
                        _rows(bn, 1), _full((1, h)), _full((1, h)), _full((1, h))]
        if li + 1 < n_layers:
            nxt = params['layers'][li + 1]
            x, p, q = pl.pallas_call(
                _final_body,
                grid=(gn,),
                in_specs=common_specs + [_full((h, h2)), _full((h, h2)), _full((1, h2))],
                out_specs=[_rows(bn, h), _rows(bn, h2), _rows(bn, h2)],
                out_shape=[jax.ShapeDtypeStruct((n, h), f32),
                           jax.ShapeDtypeStruct((n, h2), f32),
                           jax.ShapeDtypeStruct((n, h2), f32)],
            )(*common, nxt['w1'][:h], nxt['w1'][h:h2], nxt['b1'].reshape(1, h2))
        else:
            x = pl.pallas_call(
                _final_last_body,
                grid=(gn,),
                in_specs=common_specs,
                out_specs=_rows(bn, h),
                out_shape=jax.ShapeDtypeStruct((n, h), f32),
            )(*common)

    hp = params['head']
    out = pl.pallas_call(
        functools.partial(_readout_body, n),
        grid=(gn,),
        in_specs=[_rows(bn, h), _full((h2, h)), _full((1, h)),
                  _full((h, dh)), _full((1, dh)),
                  _full((dh, dout)), _full((1, dout))],
        out_specs=_full((1, dout)),
        out_shape=jax.ShapeDtypeStruct((1, dout), f32),
        scratch_shapes=[pltpu.VMEM((1, h), f32), pltpu.VMEM((1, h), f32)],
    )(x, hp['w1'], hp['b1'].reshape(1, h), hp['w2'], hp['b2'].reshape(1, dh),
      hp['w3'], hp['b3'].reshape(1, dout))
    return out


# speedup vs baseline: 2.1381x; 1.2134x over previous
import functools

import jax
import jax.numpy as jnp
from jax import lax
from jax.experimental import pallas as pl
from jax.experimental.pallas import tpu as pltpu
from jax.experimental.pallas import tpu_sc as plsc

_SQRT1_2 = 0.7071067811865476


def _gelu(x):
    return 0.5 * x * (1.0 + lax.erf(x * _SQRT1_2))


def _ln(x, g, b, eps=1e-5):
    mu = jnp.mean(x, axis=-1, keepdims=True)
    var = jnp.mean((x - mu) ** 2, axis=-1, keepdims=True)
    return (x - mu) / jnp.sqrt(var + eps) * g + b


def _enc_body(atom, ew, eb, g, bt, w1a, w1b, b1, xo, po, qo):
    h = _gelu(jnp.dot(atom[...], ew[...], preferred_element_type=jnp.float32)
              + eb[...])
    xn = _ln(h, g[...], bt[...])
    xo[...] = xn
    po[...] = jnp.dot(xn, w1a[...], preferred_element_type=jnp.float32)
    qo[...] = jnp.dot(xn, w1b[...], preferred_element_type=jnp.float32) + b1[...]


def _edge_body(e_total, be, gp, gq, attr, w1c, w2, mo):
    i = pl.program_id(0)
    gsum = gp[...] + gq[...] + attr[...] * w1c[...]
    m = _gelu(gsum)
    m2 = jnp.dot(m, w2[...], preferred_element_type=jnp.float32)
    rowid = lax.broadcasted_iota(jnp.int32, m2.shape, 0) + i * be
    mo[...] = jnp.where(rowid < e_total, m2, 0.0)


def _final_body(x, agg, deg, b2, g, bt, w1a, w1b, b1, xo, po, qo):
    t = (x[...] + jnp.concatenate([agg[0], agg[1][:, :64]], axis=-1)
         + deg[...] * b2[...])
    xn = _ln(t, g[...], bt[...])
    xo[...] = xn
    po[...] = jnp.dot(xn, w1a[...], preferred_element_type=jnp.float32)
    qo[...] = jnp.dot(xn, w1b[...], preferred_element_type=jnp.float32) + b1[...]


def _final_last_body(x, agg, deg, b2, g, bt, xo):
    t = (x[...] + jnp.concatenate([agg[0], agg[1][:, :64]], axis=-1)
         + deg[...] * b2[...])
    xo[...] = _ln(t, g[...], bt[...])


def _deg_finish_body(degp, dego):
    dego[...] = degp[0, :, 0:1] + degp[1, :, 0:1]


def _readout_body(n_total, x, hw1, hb1, hw2, hb2, hw3, hb3, out, sm, mx):
    i = pl.program_id(0)
    last = pl.num_programs(0) - 1
    blk = x[...]
    bsum = jnp.sum(blk, axis=0, keepdims=True)
    bmax = jnp.max(blk, axis=0, keepdims=True)

    @pl.when(i == 0)
    def _():
        sm[...] = bsum
        mx[...] = bmax
        out[...] = jnp.zeros_like(out)

    @pl.when(i > 0)
    def _():
        sm[...] = sm[...] + bsum
        mx[...] = jnp.maximum(mx[...], bmax)

    @pl.when(i == last)
    def _():
        ge = jnp.concatenate([sm[...] * (1.0 / n_total), mx[...]], axis=1)
        h1 = _gelu(jnp.dot(ge, hw1[...], preferred_element_type=jnp.float32)
                   + hb1[...])
        h2 = _gelu(jnp.dot(h1, hw2[...], preferred_element_type=jnp.float32)
                   + hb2[...])
        out[...] = jnp.dot(h2, hw3[...], preferred_element_type=jnp.float32) + hb3[...]


def _full(shape):
    return pl.BlockSpec(shape, lambda i: (0,) * len(shape))


def _rows(bn, d):
    return pl.BlockSpec((bn, d), lambda i: (i, 0))


_CH = 128


def _sc_gather_fn(n_chunks, epw, gch, nc, p_hbm, q_hbm, src_hbm, dst_hbm,
                  gp_hbm, gq_hbm, idx_s, idx_d, bufp, bufq, gsemp, gsemq,
                  wsemp, wsemq):
    c = lax.axis_index("c")
    s = lax.axis_index("s")
    wid = s * nc + c
    cbase = wid * n_chunks
    base = wid * epw
    pltpu.sync_copy(src_hbm.at[pl.ds(cbase, n_chunks)], idx_s)
    pltpu.sync_copy(dst_hbm.at[pl.ds(cbase, n_chunks)], idx_d)

    def g_issue(j, slot):
        pltpu.async_copy(p_hbm.at[idx_s.at[j]], bufp.at[slot], gsemp.at[slot])
        pltpu.async_copy(q_hbm.at[idx_d.at[j]], bufq.at[slot], gsemq.at[slot])

    def g_wait(j, slot):
        pltpu.make_async_copy(p_hbm.at[idx_s.at[j]], bufp.at[slot], gsemp.at[slot]).wait()
        pltpu.make_async_copy(q_hbm.at[idx_d.at[j]], bufq.at[slot], gsemq.at[slot]).wait()

    def w_issue(j, slot):
        pltpu.async_copy(bufp.at[slot], gp_hbm.at[pl.ds(base + j * gch, gch)], wsemp.at[slot])
        pltpu.async_copy(bufq.at[slot], gq_hbm.at[pl.ds(base + j * gch, gch)], wsemq.at[slot])

    def w_wait(j, slot):
        pltpu.make_async_copy(bufp.at[slot], gp_hbm.at[pl.ds(base + j * gch, gch)], wsemp.at[slot]).wait()
        pltpu.make_async_copy(bufq.at[slot], gq_hbm.at[pl.ds(base + j * gch, gch)], wsemq.at[slot]).wait()

    g_issue(0, 0)

    def body(i, _):
        slot = lax.rem(i, 2)
        other = 1 - slot

        @pl.when(i >= 1)
        def _():
            w_wait(i - 1, other)

        @pl.when(i + 1 < n_chunks)
        def _():
            g_issue(i + 1, other)

        g_wait(i, slot)
        w_issue(i, slot)
        return 0

    lax.fori_loop(0, n_chunks, body, 0)
    w_wait(n_chunks - 1, (n_chunks - 1) % 2)


def _sc_scatter_fn(n_chunks, epw, hh, rpt, m2_hbm, dst_hbm, zeros_hbm,
                   out_hbm, agg_sh, idx_d, buf, lsem, ssem):
    c = lax.axis_index("c")
    s = lax.axis_index("s")
    base = s * epw
    cbase = s * n_chunks

    pltpu.sync_copy(zeros_hbm.at[pl.ds(s * rpt, rpt)], agg_sh.at[pl.ds(s * rpt, rpt)])
    pltpu.sync_copy(dst_hbm.at[pl.ds(cbase, n_chunks)], idx_d)
    plsc.subcore_barrier()

    def l_issue(j, slot):
        pltpu.async_copy(m2_hbm.at[pl.ds(base + j * _CH, _CH), pl.ds(c * hh, hh)],
                         buf.at[slot], lsem.at[slot])

    def l_wait(j, slot):
        pltpu.make_async_copy(m2_hbm.at[pl.ds(base + j * _CH, _CH), pl.ds(c * hh, hh)],
                              buf.at[slot], lsem.at[slot]).wait()

    def s_issue(j, slot):
        pltpu.async_copy(buf.at[slot], agg_sh.at[idx_d.at[j]], ssem.at[slot], add=True)

    def s_wait(j, slot):
        pltpu.make_async_copy(buf.at[slot], agg_sh.at[idx_d.at[j]], ssem.at[slot]).wait()

    l_issue(0, 0)

    def body(i, _):
        slot = lax.rem(i, 2)
        other = 1 - slot

        @pl.when(i >= 1)
        def _():
            s_wait(i - 1, other)

        @pl.when(i + 1 < n_chunks)
        def _():
            l_issue(i + 1, other)

        l_wait(i, slot)
        s_issue(i, slot)
        return 0

    lax.fori_loop(0, n_chunks, body, 0)
    s_wait(n_chunks - 1, (n_chunks - 1) % 2)
    plsc.subcore_barrier()
    pltpu.sync_copy(agg_sh.at[pl.ds(s * rpt, rpt)], out_hbm.at[c, pl.ds(s * rpt, rpt)])


def _sc_deg_fn(n_chunks, epc, epw, rpt, dst_hbm, ones_hbm, zeros_hbm,
               out_hbm, deg_sh, idx_d, ones_v):
    c = lax.axis_index("c")
    s = lax.axis_index("s")
    base = c * epc + s * epw

    pltpu.sync_copy(zeros_hbm.at[pl.ds(s * rpt, rpt)], deg_sh.at[pl.ds(s * rpt, rpt)])
    pltpu.sync_copy(ones_hbm, ones_v)
    plsc.subcore_barrier()

    def body(i, _):
        off = base + i * _CH
        pltpu.sync_copy(dst_hbm.at[pl.ds(off, _CH)], idx_d)
        pltpu.sync_copy(ones_v, deg_sh.at[idx_d], add=True)
        return 0

    lax.fori_loop(0, n_chunks, body, 0)
    plsc.subcore_barrier()
    pltpu.sync_copy(deg_sh.at[pl.ds(s * rpt, rpt)], out_hbm.at[c, pl.ds(s * rpt, rpt)])


def kernel(atom_features, edge_index, edge_attr, params):
    n, adim = atom_features.shape
    e = edge_index.shape[1]
    h = params['enc']['w'].shape[1]
    h2 = 2 * h
    dh = params['head']['w2'].shape[1]
    dout = params['head']['w3'].shape[1]
    f32 = jnp.float32

    ncores, nsub = 2, 16
    nw = ncores * nsub
    epad = ((e + nw * _CH - 1) // (nw * _CH)) * (nw * _CH)
    epw = epad // nw
    epc = epad // ncores
    epw_sc = epc // nsub
    n_chunks = epw // _CH

    gch = 64
    src = jnp.pad(edge_index[0], (0, epad - e)).reshape(epad // gch, gch)
    dst1 = jnp.pad(edge_index[1], (0, epad - e))
    dstg = dst1.reshape(epad // gch, gch)
    dst = dst1.reshape(epad // _CH, _CH)
    attr_p = jnp.pad(edge_attr, ((0, epad - e), (0, 0)))

    npad = ((n + 8 * nsub - 1) // (8 * nsub)) * (8 * nsub)
    rpt = npad // nsub
    hh = 128
    zeros_nh = jnp.zeros((npad, hh), f32)
    zeros_n16 = jnp.zeros((npad, 16), f32)
    ones16 = jnp.zeros((_CH, 16), f32).at[:, 0].set(1.0)

    mesh = plsc.VectorSubcoreMesh(core_axis_name="c", subcore_axis_name="s")
    sc_params = pltpu.CompilerParams(use_tc_tiling_on_sc=False)

    ng = epw // gch
    sc_gather = pl.kernel(
        functools.partial(_sc_gather_fn, ng, epw, gch, ncores),
        out_type=(jax.ShapeDtypeStruct((epad, h2), f32),
                  jax.ShapeDtypeStruct((epad, h2), f32)),
        mesh=mesh,
        scratch_types=[
            pltpu.VMEM((ng, gch), jnp.int32),
            pltpu.VMEM((ng, gch), jnp.int32),
            pltpu.VMEM((2, gch, h2), f32),
            pltpu.VMEM((2, gch, h2), f32),
            pltpu.SemaphoreType.DMA((2,)),
            pltpu.SemaphoreType.DMA((2,)),
            pltpu.SemaphoreType.DMA((2,)),
            pltpu.SemaphoreType.DMA((2,)),
        ],
        compiler_params=pltpu.CompilerParams(use_tc_tiling_on_sc=True),
    )

    nchunks_sc = epad // nsub // _CH
    sc_scatter = pl.kernel(
        functools.partial(_sc_scatter_fn, nchunks_sc, epad // nsub, hh, rpt),
        out_type=jax.ShapeDtypeStruct((ncores, npad, hh), f32),
        mesh=mesh,
        scratch_types=[
            pltpu.VMEM_SHARED((npad, hh), f32),
            pltpu.VMEM((nchunks_sc, _CH), jnp.int32),
            pltpu.VMEM((2, _CH, hh), f32),
            pltpu.SemaphoreType.DMA((2,)),
            pltpu.SemaphoreType.DMA((2,)),
        ],
        compiler_params=pltpu.CompilerParams(use_tc_tiling_on_sc=True),
    )

    sc_deg = pl.kernel(
        functools.partial(_sc_deg_fn, epc // _CH // nsub, epc, epw_sc, rpt),
        out_type=jax.ShapeDtypeStruct((ncores, npad, 16), f32),
        mesh=mesh,
        scratch_types=[
            pltpu.VMEM_SHARED((npad, 16), f32),
            pltpu.VMEM((_CH,), jnp.int32),
            pltpu.VMEM((_CH, 16), f32),
        ],
        compiler_params=sc_params,
    )

    bn = 2000
    gn = n // bn
    be = 2048
    ge = epad // be

    enc = params['enc']
    lp0 = params['layers'][0]
    x, p, q = pl.pallas_call(
        _enc_body,
        grid=(gn,),
        in_specs=[_rows(bn, adim), _full((adim, h)), _full((1, h)),
                  _full((1, h)), _full((1, h)),
                  _full((h, h2)), _full((h, h2)), _full((1, h2))],
        out_specs=[_rows(bn, h), _rows(bn, h2), _rows(bn, h2)],
        out_shape=[jax.ShapeDtypeStruct((n, h), f32),
                   jax.ShapeDtypeStruct((n, h2), f32),
                   jax.ShapeDtypeStruct((n, h2), f32)],
    )(atom_features, enc['w'], enc['b'].reshape(1, h),
      enc['g'].reshape(1, h), enc['bt'].reshape(1, h),
      lp0['w1'][:h], lp0['w1'][h:h2], lp0['b1'].reshape(1, h2))

    degp = sc_deg(dst1, ones16, zeros_n16)
    deg = pl.pallas_call(
        _deg_finish_body,
        grid=(gn,),
        in_specs=[pl.BlockSpec((ncores, bn, 16), lambda i: (0, i, 0))],
        out_specs=_rows(bn, 1),
        out_shape=jax.ShapeDtypeStruct((n, 1), f32),
    )(degp)

    n_layers = len(params['layers'])
    for li, lp in enumerate(params['layers']):
        gp, gq = sc_gather(p, q, src, dstg)
        m2 = pl.pallas_call(
            functools.partial(_edge_body, e, be),
            grid=(ge,),
            in_specs=[_rows(be, h2), _rows(be, h2), _rows(be, 1),
                      _full((1, h2)), _full((h2, 2 * hh))],
            out_specs=_rows(be, 2 * hh),
            out_shape=jax.ShapeDtypeStruct((epad, 2 * hh), f32),
        )(gp, gq, attr_p, lp['w1'][h2:h2 + 1],
          jnp.pad(lp['w2'], ((0, 0), (0, 2 * hh - h))))
        aggp = sc_scatter(m2, dst, zeros_nh)

        common = (x, aggp, deg, lp['b2'].reshape(1, h),
                  lp['g'].reshape(1, h), lp['bt'].reshape(1, h))
        common_specs = [_rows(bn, h), pl.BlockSpec((ncores, bn, hh), lambda i: (0, i, 0)),
                        _rows(bn, 1), _full((1, h)), _full((1, h)), _full((1, h))]
        if li + 1 < n_layers:
            nxt = params['layers'][li + 1]
            x, p, q = pl.pallas_call(
                _final_body,
                grid=(gn,),
                in_specs=common_specs + [_full((h, h2)), _full((h, h2)), _full((1, h2))],
                out_specs=[_rows(bn, h), _rows(bn, h2), _rows(bn, h2)],
                out_shape=[jax.ShapeDtypeStruct((n, h), f32),
                           jax.ShapeDtypeStruct((n, h2), f32),
                           jax.ShapeDtypeStruct((n, h2), f32)],
            )(*common, nxt['w1'][:h], nxt['w1'][h:h2], nxt['b1'].reshape(1, h2))
        else:
            x = pl.pallas_call(
                _final_last_body,
                grid=(gn,),
                in_specs=common_specs,
                out_specs=_rows(bn, h),
                out_shape=jax.ShapeDtypeStruct((n, h), f32),
            )(*common)

    hp = params['head']
    out = pl.pallas_call(
        functools.partial(_readout_body, n),
        grid=(gn,),
        in_specs=[_rows(bn, h), _full((h2, h)), _full((1, h)),
                  _full((h, dh)), _full((1, dh)),
                  _full((dh, dout)), _full((1, dout))],
        out_specs=_full((1, dout)),
        out_shape=jax.ShapeDtypeStruct((1, dout), f32),
        scratch_shapes=[pltpu.VMEM((1, h), f32), pltpu.VMEM((1, h), f32)],
    )(x, hp['w1'], hp['b1'].reshape(1, h), hp['w2'], hp['b2'].reshape(1, dh),
      hp['w3'], hp['b3'].reshape(1, dout))
    return out
